# baseline (speedup 1.0000x reference)
import functools

import jax
import jax.numpy as jnp
from jax import lax
from jax.experimental import pallas as pl
from jax.experimental.pallas import tpu as pltpu
from jax.experimental.pallas import tpu_sc as plsc

N = 10000
E = 320000
DIN = 128
BIG = 32
HID = 32
NF = 6

NC = 2
NS = 16
NW = NC * NS
EW = E // NW
CH = 125
NCH = EW // CH
CW = 16
SB = 1000
NSG = EW // SB
CPS = SB // CH
RB = 1250
RPT = 624
ZR = N // NS

_SC_MESH = dict(core_axis_name="c", subcore_axis_name="s",
                num_cores=NC, num_subcores=NS)
_SC_PARAMS = pltpu.CompilerParams(use_tc_tiling_on_sc=False)


def _drain(acc, out_hbm, cid, sid):
    last = N - (NS - 1) * RPT

    @pl.when(sid < NS - 1)
    def _():
        pltpu.sync_copy(acc.at[pl.ds(sid * RPT, RPT)],
                        out_hbm.at[cid, pl.ds(sid * RPT, RPT)])

    @pl.when(sid == NS - 1)
    def _():
        pltpu.sync_copy(acc.at[pl.ds((NS - 1) * RPT, last)],
                        out_hbm.at[cid, pl.ds((NS - 1) * RPT, last)])


def _zero_acc(zrow, acc, sid, width):
    def z(i, c):
        zrow[i, :] = jnp.zeros((width,), jnp.float32)
        return c
    lax.fori_loop(0, ZR, z, 0)
    pltpu.sync_copy(zrow, acc.at[pl.ds(sid * ZR, ZR)])


def _pq_body(x_ref, w1_ref, p_ref, q_ref):
    x = x_ref[...]
    wa = w1_ref[0:DIN, :] - w1_ref[DIN:2 * DIN, :]
    wb = w1_ref[DIN:2 * DIN, :]
    p_ref[...] = jnp.dot(x, wa, preferred_element_type=jnp.float32
                         ).astype(jnp.bfloat16)
    q_ref[...] = jnp.dot(x, wb, preferred_element_type=jnp.float32
                         ).astype(jnp.bfloat16)


def _pq(x, w1):
    nb = 10
    blk = N // nb
    return pl.pallas_call(
        _pq_body,
        grid=(nb,),
        in_specs=[
            pl.BlockSpec((blk, DIN), lambda i: (i, 0)),
            pl.BlockSpec((2 * DIN, BIG), lambda i: (0, 0)),
        ],
        out_specs=[
            pl.BlockSpec((blk, BIG), lambda i: (i, 0)),
            pl.BlockSpec((blk, BIG), lambda i: (i, 0)),
        ],
        out_shape=[
            jax.ShapeDtypeStruct((N, BIG), jnp.bfloat16),
            jax.ShapeDtypeStruct((N, BIG), jnp.bfloat16),
        ],
    )(x, w1)


def _sc_gather_body(p_hbm, q_hbm, src_hbm, dst_hbm, hp_hbm, hq_hbm, cnt_hbm,
                    sidx, didx, p0, q0, p1, q1, ones_v, zrow, acc_c,
                    sem_p, sem_q, sem_c, sem_s0, sem_s1):
    cid = lax.axis_index("c")
    sid = lax.axis_index("s")
    wid = sid * NC + cid

    _zero_acc(zrow, acc_c, sid, CW)

    pltpu.sync_copy(src_hbm.at[wid], sidx)
    pltpu.sync_copy(dst_hbm.at[wid], didx)

    def fill_ones(i, c):
        ones_v[i, :] = jnp.full((CW,), 1.0, jnp.float32)
        return c
    lax.fori_loop(0, CH, fill_ones, 0)

    plsc.subcore_barrier()

    def one_stage(s, bufp, bufq, st_sem):
        for j in range(CPS):
            pltpu.async_copy(p_hbm.at[didx.at[s * CPS + j]],
                             bufp.at[pl.ds(j * CH, CH)], sem_p)
        for j in range(CPS):
            pltpu.async_copy(q_hbm.at[sidx.at[s * CPS + j]],
                             bufq.at[pl.ds(j * CH, CH)], sem_q)
        for j in range(CPS):
            pltpu.async_copy(ones_v, acc_c.at[didx.at[s * CPS + j]],
                             sem_c, add=True)
        for j in range(CPS):
            pltpu.make_async_copy(p_hbm.at[didx.at[s * CPS + j]],
                                  bufp.at[pl.ds(j * CH, CH)], sem_p).wait()
        for j in range(CPS):
            pltpu.make_async_copy(q_hbm.at[sidx.at[s * CPS + j]],
                                  bufq.at[pl.ds(j * CH, CH)], sem_q).wait()
        pltpu.async_copy(bufp, hp_hbm.at[wid, pl.ds(s * SB, SB)], st_sem)
        pltpu.async_copy(bufq, hq_hbm.at[wid, pl.ds(s * SB, SB)], st_sem)
        for j in range(CPS):
            pltpu.make_async_copy(ones_v, acc_c.at[didx.at[s * CPS + j]],
                                  sem_c).wait()

    def wait_stores(bufp, bufq, st_sem):
        pltpu.make_async_copy(bufp, hp_hbm.at[wid, pl.ds(0, SB)], st_sem).wait()
        pltpu.make_async_copy(bufq, hq_hbm.at[wid, pl.ds(0, SB)], st_sem).wait()

    def pair_loop(t, c):
        s0 = t * 2
        @pl.when(t > 0)
        def _():
            wait_stores(p0, q0, sem_s0)
        one_stage(s0, p0, q0, sem_s0)

        @pl.when(t > 0)
        def _():
            wait_stores(p1, q1, sem_s1)
        one_stage(s0 + 1, p1, q1, sem_s1)
        return c
    lax.fori_loop(0, NSG // 2, pair_loop, 0)
    wait_stores(p0, q0, sem_s0)
    wait_stores(p1, q1, sem_s1)

    plsc.subcore_barrier()
    _drain(acc_c, cnt_hbm, cid, sid)


def _sc_gather(p, q, srcr, dstr):
    mesh = plsc.VectorSubcoreMesh(**_SC_MESH)
    f = functools.partial(
        pl.kernel,
        mesh=mesh,
        compiler_params=_SC_PARAMS,
        out_type=[
            jax.ShapeDtypeStruct((NW, EW, BIG), jnp.bfloat16),
            jax.ShapeDtypeStruct((NW, EW, BIG), jnp.bfloat16),
            jax.ShapeDtypeStruct((NC, N, CW), jnp.float32),
        ],
        scratch_types=[
            pltpu.VMEM((NCH, CH), jnp.int32),
            pltpu.VMEM((NCH, CH), jnp.int32),
            pltpu.VMEM((SB, BIG), jnp.bfloat16),
            pltpu.VMEM((SB, BIG), jnp.bfloat16),
            pltpu.VMEM((SB, BIG), jnp.bfloat16),
            pltpu.VMEM((SB, BIG), jnp.bfloat16),
            pltpu.VMEM((CH, CW), jnp.float32),
            pltpu.VMEM((ZR, CW), jnp.float32),
            pltpu.VMEM_SHARED((N, CW), jnp.float32),
            pltpu.SemaphoreType.DMA,
            pltpu.SemaphoreType.DMA,
            pltpu.SemaphoreType.DMA,
            pltpu.SemaphoreType.DMA,
            pltpu.SemaphoreType.DMA,
        ],
    )(_sc_gather_body)
    return f(p, q, srcr, dstr)


def _mlp_body(hp_ref, hq_ref, b1_ref, w2_ref, b2_ref, w3_ref, b3_ref, o_ref):
    r = 128 // BIG
    w2 = w2_ref[...]
    w3 = w3_ref[...]
    eye = jnp.eye(r, dtype=jnp.float32)
    w2bd = jnp.kron(eye, w2)
    w3bd = jnp.kron(eye, w3)
    h = (hp_ref[...].astype(jnp.float32) + hq_ref[...].astype(jnp.float32)
         + b1_ref[...])
    t = jnp.maximum(h, 0.0)
    t = jnp.maximum(
        jnp.dot(t, w2bd, preferred_element_type=jnp.float32) + b2_ref[...], 0.0)
    o_ref[...] = jnp.maximum(
        jnp.dot(t, w3bd, preferred_element_type=jnp.float32) + b3_ref[...], 0.0)


def _mlp(hp, hq, b1, w2, b2, w3, b3):
    lw = 128
    e4 = E * BIG // lw
    be = 8000
    nb = e4 // be
    r = lw // BIG
    b1t = jnp.tile(b1, (1, r))
    b2t = jnp.tile(b2, (1, r))
    b3t = jnp.tile(b3, (1, r))
    out = pl.pallas_call(
        _mlp_body,
        grid=(nb,),
        in_specs=[
            pl.BlockSpec((be, lw), lambda i: (i, 0)),
            pl.BlockSpec((be, lw), lambda i: (i, 0)),
            pl.BlockSpec((1, lw), lambda i: (0, 0)),
            pl.BlockSpec((BIG, BIG), lambda i: (0, 0)),
            pl.BlockSpec((1, lw), lambda i: (0, 0)),
            pl.BlockSpec((BIG, BIG), lambda i: (0, 0)),
            pl.BlockSpec((1, lw), lambda i: (0, 0)),
        ],
        out_specs=pl.BlockSpec((be, lw), lambda i: (i, 0)),
        out_shape=jax.ShapeDtypeStruct((e4, lw), jnp.float32),
    )(hp.reshape(e4, lw), hq.reshape(e4, lw), b1t, w2, b2t, w3, b3t)
    return out.reshape(E, BIG)


def _sc_scatter_body(msg_hbm, dst_hbm, sum_hbm,
                     didx, mrows0, mrows1, zrow, acc_s,
                     sem_l0, sem_l1, sem_sc):
    cid = lax.axis_index("c")
    sid = lax.axis_index("s")
    wid = sid * NC + cid

    _zero_acc(zrow, acc_s, sid, BIG)

    pltpu.sync_copy(dst_hbm.at[wid], didx)
    plsc.subcore_barrier()

    nj = RB // CH
    nstg = EW // RB

    def load(b, buf, lsem):
        pltpu.async_copy(msg_hbm.at[wid, pl.ds(b * RB, RB)], buf, lsem)

    def scatter_stage(b, buf, lsem):
        pltpu.make_async_copy(msg_hbm.at[wid, pl.ds(0, RB)], buf, lsem).wait()
        for j in range(nj):
            pltpu.async_copy(buf.at[pl.ds(j * CH, CH)],
                             acc_s.at[didx.at[b * nj + j]], sem_sc, add=True)
        for j in range(nj):
            pltpu.make_async_copy(buf.at[pl.ds(j * CH, CH)],
                                  acc_s.at[didx.at[b * nj + j]], sem_sc).wait()

    load(0, mrows0, sem_l0)

    def pair(t, c):
        b0 = t * 2

        @pl.when(b0 + 1 < nstg)
        def _():
            load(b0 + 1, mrows1, sem_l1)
        scatter_stage(b0, mrows0, sem_l0)

        @pl.when(b0 + 2 < nstg)
        def _():
            load(b0 + 2, mrows0, sem_l0)

        @pl.when(b0 + 1 < nstg)
        def _():
            scatter_stage(b0 + 1, mrows1, sem_l1)
        return c
    lax.fori_loop(0, (nstg + 1) // 2, pair, 0)

    plsc.subcore_barrier()
    _drain(acc_s, sum_hbm, cid, sid)


def _sc_scatter(msgr, dstr):
    mesh = plsc.VectorSubcoreMesh(**_SC_MESH)
    f = functools.partial(
        pl.kernel,
        mesh=mesh,
        compiler_params=_SC_PARAMS,
        out_type=jax.ShapeDtypeStruct((NC, N, BIG), jnp.float32),
        scratch_types=[
            pltpu.VMEM((NCH, CH), jnp.int32),
            pltpu.VMEM((RB, BIG), jnp.float32),
            pltpu.VMEM((RB, BIG), jnp.float32),
            pltpu.VMEM((ZR, BIG), jnp.float32),
            pltpu.VMEM_SHARED((N, BIG), jnp.float32),
            pltpu.SemaphoreType.DMA,
            pltpu.SemaphoreType.DMA,
            pltpu.SemaphoreType.DMA,
        ],
    )(_sc_scatter_body)
    return f(msgr, dstr)


def _flow_body(s_ref, c_ref, eps_ref,
               wmu_ref, bmu_ref, wvar_ref, bvar_ref,
               wu_ref, bu_ref, ww_ref, bw_ref, wb_ref, bb_ref,
               mu_ref, lv_ref, z0_ref, zk_ref, ldj_ref):
    cnt = c_ref[0][:, 0:1] + c_ref[1][:, 0:1]
    h = (s_ref[0] + s_ref[1]) / jnp.maximum(cnt, 1.0)
    hT = jnp.transpose(h, (1, 0))

    def head(w_ref, b_ref):
        wT = jnp.transpose(w_ref[...], (1, 0))
        bT = jnp.transpose(b_ref[...], (1, 0))
        return jnp.dot(wT, hT, preferred_element_type=jnp.float32) + bT

    mu = head(wmu_ref, bmu_ref)
    lv = head(wvar_ref, bvar_ref)
    uu = head(wu_ref, bu_ref)
    ww = head(ww_ref, bw_ref)
    bf = head(wb_ref, bb_ref)

    epsT = jnp.transpose(eps_ref[...], (1, 0))
    z = mu + epsT * jnp.exp(0.5 * lv)
    mu_ref[...] = jnp.transpose(mu, (1, 0))
    lv_ref[...] = jnp.transpose(lv, (1, 0))
    z0_ref[...] = jnp.transpose(z, (1, 0))

    ldj = jnp.zeros_like(bf[0:1])
    for k in range(NF):
        uk = uu[k * HID:(k + 1) * HID]
        wk = ww[k * HID:(k + 1) * HID]
        bk = bf[k:k + 1]
        uw = jnp.sum(wk * uk, axis=0, keepdims=True)
        m_uw = -1.0 + jnp.logaddexp(uw, 0.0)
        wns = jnp.sum(wk * wk, axis=0, keepdims=True)
        u_hat = uk + ((m_uw - uw) / wns) * wk
        wzb = jnp.sum(wk * z, axis=0, keepdims=True) + bk
        t = jnp.tanh(wzb)
        z = z + u_hat * t
        wu_dot = jnp.sum(wk * u_hat, axis=0, keepdims=True)
        ldj = ldj + jnp.log(jnp.abs(1.0 + (1.0 - t * t) * wu_dot))

    zk_ref[...] = jnp.transpose(z, (1, 0))
    ldj_ref[...] = jnp.transpose(ldj, (1, 0))


def _flow(sums, cnts, eps, wmu, bmu, wvar, bvar, wu, bu, ww, bw, wb, bb):
    nb = 10
    blk = N // nb
    full = lambda r, c: pl.BlockSpec((r, c), lambda i: (0, 0))
    return pl.pallas_call(
        _flow_body,
        grid=(nb,),
        in_specs=[
            pl.BlockSpec((NC, blk, BIG), lambda i: (0, i, 0)),
            pl.BlockSpec((NC, blk, CW), lambda i: (0, i, 0)),
            pl.BlockSpec((blk, HID), lambda i: (i, 0)),
            full(BIG, HID), full(1, HID),
            full(BIG, HID), full(1, HID),
            full(BIG, NF * HID), full(1, NF * HID),
            full(BIG, NF * HID), full(1, NF * HID),
            full(BIG, NF), full(1, NF),
        ],
        out_specs=[
            pl.BlockSpec((blk, HID), lambda i: (i, 0)),
            pl.BlockSpec((blk, HID), lambda i: (i, 0)),
            pl.BlockSpec((blk, HID), lambda i: (i, 0)),
            pl.BlockSpec((blk, HID), lambda i: (i, 0)),
            pl.BlockSpec((blk, 1), lambda i: (i, 0)),
        ],
        out_shape=[
            jax.ShapeDtypeStruct((N, HID), jnp.float32),
            jax.ShapeDtypeStruct((N, HID), jnp.float32),
            jax.ShapeDtypeStruct((N, HID), jnp.float32),
            jax.ShapeDtypeStruct((N, HID), jnp.float32),
            jax.ShapeDtypeStruct((N, 1), jnp.float32),
        ],
    )(sums, cnts, eps, wmu, bmu, wvar, bvar, wu, bu, ww, bw, wb, bb)


def kernel(x, edge_index, We1, be1, We2, be2, We3, be3, Wmu, bmu, Wvar, bvar,
           Wu, bu, Ww, bw, Wb, bb, Wd1, bd1, Wd2, bd2, Wd3, bd3):
    src = edge_index[0]
    dst = edge_index[1]

    p, q = _pq(x, We1)

    srcr = src.reshape(NW, NCH, CH)
    dstr = dst.reshape(NW, NCH, CH)

    hp, hq, cnt = _sc_gather(p, q, srcr, dstr)
    msg = _mlp(hp.reshape(E, BIG), hq.reshape(E, BIG), be1.reshape(1, BIG),
               We2, be2.reshape(1, BIG), We3, be3.reshape(1, BIG))
    sums = _sc_scatter(msg.reshape(NW, EW, BIG), dstr)

    eps = jax.random.normal(jax.random.key(42), (N, HID), dtype=jnp.float32)
    mu, lv, z0, zk, ldj = _flow(
        sums, cnt, eps,
        Wmu, bmu.reshape(1, HID), Wvar, bvar.reshape(1, HID),
        Wu, bu.reshape(1, NF * HID), Ww, bw.reshape(1, NF * HID),
        Wb, bb.reshape(1, NF))

    return (zk, mu, lv, ldj.reshape(N), z0, zk)


# speedup vs baseline: 1.6972x; 1.6972x over previous
import functools

import jax
import jax.numpy as jnp
from jax import lax
from jax.experimental import pallas as pl
from jax.experimental.pallas import tpu as pltpu
from jax.experimental.pallas import tpu_sc as plsc

N = 10000
E = 320000
DIN = 128
BIG = 32
HID = 32
NF = 6

NC = 2
NS = 16
NW = NC * NS
EW = E // NW
CH = 125
NCH = EW // CH
CW = 16
SB = 1000
NSG = EW // SB
CPS = SB // CH
RB = 1250
RPT = 624
ZR = N // NS

_SC_MESH = dict(core_axis_name="c", subcore_axis_name="s",
                num_cores=NC, num_subcores=NS)
_SC_PARAMS = pltpu.CompilerParams(use_tc_tiling_on_sc=False)


def _drain(acc, out_hbm, cid, sid):
    last = N - (NS - 1) * RPT

    @pl.when(sid < NS - 1)
    def _():
        pltpu.sync_copy(acc.at[pl.ds(sid * RPT, RPT)],
                        out_hbm.at[cid, pl.ds(sid * RPT, RPT)])

    @pl.when(sid == NS - 1)
    def _():
        pltpu.sync_copy(acc.at[pl.ds((NS - 1) * RPT, last)],
                        out_hbm.at[cid, pl.ds((NS - 1) * RPT, last)])


def _zero_acc(zrow, acc, sid, width):
    def z(i, c):
        zrow[i, :] = jnp.zeros((width,), jnp.float32)
        return c
    lax.fori_loop(0, ZR, z, 0)
    pltpu.sync_copy(zrow, acc.at[pl.ds(sid * ZR, ZR)])


def _pq_body(x_ref, w1_ref, p_ref, q_ref):
    x = x_ref[...]
    wa = w1_ref[0:DIN, :] - w1_ref[DIN:2 * DIN, :]
    wb = w1_ref[DIN:2 * DIN, :]
    p_ref[...] = jnp.dot(x, wa, preferred_element_type=jnp.float32)
    q_ref[...] = jnp.dot(x, wb, preferred_element_type=jnp.float32)


def _pq(x, w1):
    nb = 10
    blk = N // nb
    return pl.pallas_call(
        _pq_body,
        grid=(nb,),
        in_specs=[
            pl.BlockSpec((blk, DIN), lambda i: (i, 0)),
            pl.BlockSpec((2 * DIN, BIG), lambda i: (0, 0)),
        ],
        out_specs=[
            pl.BlockSpec((blk, BIG), lambda i: (i, 0)),
            pl.BlockSpec((blk, BIG), lambda i: (i, 0)),
        ],
        out_shape=[
            jax.ShapeDtypeStruct((N, BIG), jnp.float32),
            jax.ShapeDtypeStruct((N, BIG), jnp.float32),
        ],
    )(x, w1)


def _sc_gather_body(p_hbm, q_hbm, src_hbm, dst_hbm, h1_hbm, cnt_hbm,
                    sidx, didx, stage0, stage1, ones_v, zrow, acc_c,
                    sem_p0, sem_p1, sem_q, sem_c, sem_s0, sem_s1):
    cid = lax.axis_index("c")
    sid = lax.axis_index("s")
    wid = sid * NC + cid

    _zero_acc(zrow, acc_c, sid, CW)

    pltpu.sync_copy(src_hbm.at[wid], sidx)
    pltpu.sync_copy(dst_hbm.at[wid], didx)

    def fill_ones(i, c):
        ones_v[i, :] = jnp.full((CW,), 1.0, jnp.float32)
        return c
    lax.fori_loop(0, CH, fill_ones, 0)

    plsc.subcore_barrier()

    def fire_p(s, buf, psem):
        for j in range(CPS):
            pltpu.async_copy(p_hbm.at[didx.at[s * CPS + j]],
                             buf.at[pl.ds(j * CH, CH)], psem)

    def wait_p(s, buf, psem):
        for j in range(CPS):
            pltpu.make_async_copy(p_hbm.at[didx.at[s * CPS + j]],
                                  buf.at[pl.ds(j * CH, CH)], psem).wait()

    def fire_q(s, buf):
        for j in range(CPS):
            pltpu.async_copy(q_hbm.at[sidx.at[s * CPS + j]],
                             buf.at[pl.ds(j * CH, CH)], sem_q, add=True)

    def wait_q(s, buf):
        for j in range(CPS):
            pltpu.make_async_copy(q_hbm.at[sidx.at[s * CPS + j]],
                                  buf.at[pl.ds(j * CH, CH)], sem_q).wait()

    def fire_counts(s):
        for j in range(CPS):
            pltpu.async_copy(ones_v, acc_c.at[didx.at[s * CPS + j]],
                             sem_c, add=True)

    def wait_counts(s):
        for j in range(CPS):
            pltpu.make_async_copy(ones_v, acc_c.at[didx.at[s * CPS + j]],
                                  sem_c).wait()

    def store(s, buf, st_sem):
        pltpu.async_copy(buf, h1_hbm.at[wid, pl.ds(s * SB, SB)], st_sem)

    def wait_store(buf, st_sem):
        pltpu.make_async_copy(buf, h1_hbm.at[wid, pl.ds(0, SB)], st_sem).wait()

    fire_p(0, stage0, sem_p0)
    npair = NSG // 2

    def pair_loop(t, c):
        s0 = t * 2
        fire_counts(s0)

        @pl.when(t > 0)
        def _():
            wait_store(stage1, sem_s1)
        fire_p(s0 + 1, stage1, sem_p1)
        wait_p(s0, stage0, sem_p0)
        fire_q(s0, stage0)
        fire_counts(s0 + 1)
        wait_q(s0, stage0)
        store(s0, stage0, sem_s0)
        wait_p(s0 + 1, stage1, sem_p1)
        fire_q(s0 + 1, stage1)

        @pl.when(t + 1 < npair)
        def _():
            wait_store(stage0, sem_s0)
            fire_p(s0 + 2, stage0, sem_p0)
        wait_q(s0 + 1, stage1)
        store(s0 + 1, stage1, sem_s1)
        wait_counts(s0)
        wait_counts(s0 + 1)
        return c
    lax.fori_loop(0, npair, pair_loop, 0)
    wait_store(stage0, sem_s0)
    wait_store(stage1, sem_s1)

    plsc.subcore_barrier()
    _drain(acc_c, cnt_hbm, cid, sid)


def _sc_gather(p, q, srcr, dstr):
    mesh = plsc.VectorSubcoreMesh(**_SC_MESH)
    f = functools.partial(
        pl.kernel,
        mesh=mesh,
        compiler_params=_SC_PARAMS,
        out_type=[
            jax.ShapeDtypeStruct((NW, EW, BIG), jnp.float32),
            jax.ShapeDtypeStruct((NC, N, CW), jnp.float32),
        ],
        scratch_types=[
            pltpu.VMEM((NCH, CH), jnp.int32),
            pltpu.VMEM((NCH, CH), jnp.int32),
            pltpu.VMEM((SB, BIG), jnp.float32),
            pltpu.VMEM((SB, BIG), jnp.float32),
            pltpu.VMEM((CH, CW), jnp.float32),
            pltpu.VMEM((ZR, CW), jnp.float32),
            pltpu.VMEM_SHARED((N, CW), jnp.float32),
            pltpu.SemaphoreType.DMA,
            pltpu.SemaphoreType.DMA,
            pltpu.SemaphoreType.DMA,
            pltpu.SemaphoreType.DMA,
            pltpu.SemaphoreType.DMA,
            pltpu.SemaphoreType.DMA,
        ],
    )(_sc_gather_body)
    return f(p, q, srcr, dstr)


def _mlp_body(h_ref, b1_ref, w2_ref, b2_ref, w3_ref, b3_ref, o_ref):
    r = 128 // BIG
    eye = jnp.eye(r, dtype=jnp.float32)
    w2bd = jnp.kron(eye, w2_ref[...])
    w3bd = jnp.kron(eye, w3_ref[...])
    t = jnp.maximum(h_ref[...] + b1_ref[...], 0.0)
    t = jnp.maximum(
        jnp.dot(t, w2bd, preferred_element_type=jnp.float32) + b2_ref[...], 0.0)
    o_ref[...] = jnp.maximum(
        jnp.dot(t, w3bd, preferred_element_type=jnp.float32) + b3_ref[...], 0.0)


def _mlp(h1, b1, w2, b2, w3, b3):
    lw = 128
    e4 = E * BIG // lw
    be = 8000
    nb = e4 // be
    r = lw // BIG
    b1t = jnp.tile(b1, (1, r))
    b2t = jnp.tile(b2, (1, r))
    b3t = jnp.tile(b3, (1, r))
    out = pl.pallas_call(
        _mlp_body,
        grid=(nb,),
        in_specs=[
            pl.BlockSpec((be, lw), lambda i: (i, 0)),
            pl.BlockSpec((1, lw), lambda i: (0, 0)),
            pl.BlockSpec((BIG, BIG), lambda i: (0, 0)),
            pl.BlockSpec((1, lw), lambda i: (0, 0)),
            pl.BlockSpec((BIG, BIG), lambda i: (0, 0)),
            pl.BlockSpec((1, lw), lambda i: (0, 0)),
        ],
        out_specs=pl.BlockSpec((be, lw), lambda i: (i, 0)),
        out_shape=jax.ShapeDtypeStruct((e4, lw), jnp.float32),
    )(h1.reshape(e4, lw), b1t, w2, b2t, w3, b3t)
    return out.reshape(E, BIG)


def _sc_scatter_body(msg_hbm, dst_hbm, sum_hbm,
                     didx, mrows0, mrows1, zrow, acc_s,
                     sem_l0, sem_l1, sem_sc):
    cid = lax.axis_index("c")
    sid = lax.axis_index("s")
    wid = sid * NC + cid

    _zero_acc(zrow, acc_s, sid, BIG)

    pltpu.sync_copy(dst_hbm.at[wid], didx)
    plsc.subcore_barrier()

    nj = RB // CH
    nstg = EW // RB

    def load(b, buf, lsem):
        pltpu.async_copy(msg_hbm.at[wid, pl.ds(b * RB, RB)], buf, lsem)

    def scatter_stage(b, buf, lsem):
        pltpu.make_async_copy(msg_hbm.at[wid, pl.ds(0, RB)], buf, lsem).wait()
        for j in range(nj):
            pltpu.async_copy(buf.at[pl.ds(j * CH, CH)],
                             acc_s.at[didx.at[b * nj + j]], sem_sc, add=True)
        for j in range(nj):
            pltpu.make_async_copy(buf.at[pl.ds(j * CH, CH)],
                                  acc_s.at[didx.at[b * nj + j]], sem_sc).wait()

    load(0, mrows0, sem_l0)

    def pair(t, c):
        b0 = t * 2

        @pl.when(b0 + 1 < nstg)
        def _():
            load(b0 + 1, mrows1, sem_l1)
        scatter_stage(b0, mrows0, sem_l0)

        @pl.when(b0 + 2 < nstg)
        def _():
            load(b0 + 2, mrows0, sem_l0)

        @pl.when(b0 + 1 < nstg)
        def _():
            scatter_stage(b0 + 1, mrows1, sem_l1)
        return c
    lax.fori_loop(0, (nstg + 1) // 2, pair, 0)

    plsc.subcore_barrier()
    _drain(acc_s, sum_hbm, cid, sid)


def _sc_scatter(msgr, dstr):
    mesh = plsc.VectorSubcoreMesh(**_SC_MESH)
    f = functools.partial(
        pl.kernel,
        mesh=mesh,
        compiler_params=_SC_PARAMS,
        out_type=jax.ShapeDtypeStruct((NC, N, BIG), jnp.float32),
        scratch_types=[
            pltpu.VMEM((NCH, CH), jnp.int32),
            pltpu.VMEM((RB, BIG), jnp.float32),
            pltpu.VMEM((RB, BIG), jnp.float32),
            pltpu.VMEM((ZR, BIG), jnp.float32),
            pltpu.VMEM_SHARED((N, BIG), jnp.float32),
            pltpu.SemaphoreType.DMA,
            pltpu.SemaphoreType.DMA,
            pltpu.SemaphoreType.DMA,
        ],
    )(_sc_scatter_body)
    return f(msgr, dstr)


def _flow_body(s_ref, c_ref, eps_ref,
               wmu_ref, bmu_ref, wvar_ref, bvar_ref,
               wu_ref, bu_ref, ww_ref, bw_ref, wb_ref, bb_ref,
               mu_ref, lv_ref, z0_ref, zk_ref, ldj_ref):
    cnt = c_ref[0][:, 0:1] + c_ref[1][:, 0:1]
    h = (s_ref[0] + s_ref[1]) / jnp.maximum(cnt, 1.0)
    hT = jnp.transpose(h, (1, 0))

    def head(w_ref, b_ref):
        wT = jnp.transpose(w_ref[...], (1, 0))
        bT = jnp.transpose(b_ref[...], (1, 0))
        return jnp.dot(wT, hT, preferred_element_type=jnp.float32) + bT

    mu = head(wmu_ref, bmu_ref)
    lv = head(wvar_ref, bvar_ref)
    uu = head(wu_ref, bu_ref)
    ww = head(ww_ref, bw_ref)
    bf = head(wb_ref, bb_ref)

    epsT = jnp.transpose(eps_ref[...], (1, 0))
    z = mu + epsT * jnp.exp(0.5 * lv)
    mu_ref[...] = jnp.transpose(mu, (1, 0))
    lv_ref[...] = jnp.transpose(lv, (1, 0))
    z0_ref[...] = jnp.transpose(z, (1, 0))

    ldj = jnp.zeros_like(bf[0:1])
    for k in range(NF):
        uk = uu[k * HID:(k + 1) * HID]
        wk = ww[k * HID:(k + 1) * HID]
        bk = bf[k:k + 1]
        uw = jnp.sum(wk * uk, axis=0, keepdims=True)
        m_uw = -1.0 + jnp.logaddexp(uw, 0.0)
        wns = jnp.sum(wk * wk, axis=0, keepdims=True)
        u_hat = uk + ((m_uw - uw) / wns) * wk
        wzb = jnp.sum(wk * z, axis=0, keepdims=True) + bk
        t = jnp.tanh(wzb)
        z = z + u_hat * t
        wu_dot = jnp.sum(wk * u_hat, axis=0, keepdims=True)
        ldj = ldj + jnp.log(jnp.abs(1.0 + (1.0 - t * t) * wu_dot))

    zk_ref[...] = jnp.transpose(z, (1, 0))
    ldj_ref[...] = jnp.transpose(ldj, (1, 0))


def _flow(sums, cnts, eps, wmu, bmu, wvar, bvar, wu, bu, ww, bw, wb, bb):
    nb = 10
    blk = N // nb
    full = lambda r, c: pl.BlockSpec((r, c), lambda i: (0, 0))
    return pl.pallas_call(
        _flow_body,
        grid=(nb,),
        in_specs=[
            pl.BlockSpec((NC, blk, BIG), lambda i: (0, i, 0)),
            pl.BlockSpec((NC, blk, CW), lambda i: (0, i, 0)),
            pl.BlockSpec((blk, HID), lambda i: (i, 0)),
            full(BIG, HID), full(1, HID),
            full(BIG, HID), full(1, HID),
            full(BIG, NF * HID), full(1, NF * HID),
            full(BIG, NF * HID), full(1, NF * HID),
            full(BIG, NF), full(1, NF),
        ],
        out_specs=[
            pl.BlockSpec((blk, HID), lambda i: (i, 0)),
            pl.BlockSpec((blk, HID), lambda i: (i, 0)),
            pl.BlockSpec((blk, HID), lambda i: (i, 0)),
            pl.BlockSpec((blk, HID), lambda i: (i, 0)),
            pl.BlockSpec((blk, 1), lambda i: (i, 0)),
        ],
        out_shape=[
            jax.ShapeDtypeStruct((N, HID), jnp.float32),
            jax.ShapeDtypeStruct((N, HID), jnp.float32),
            jax.ShapeDtypeStruct((N, HID), jnp.float32),
            jax.ShapeDtypeStruct((N, HID), jnp.float32),
            jax.ShapeDtypeStruct((N, 1), jnp.float32),
        ],
    )(sums, cnts, eps, wmu, bmu, wvar, bvar, wu, bu, ww, bw, wb, bb)


def kernel(x, edge_index, We1, be1, We2, be2, We3, be3, Wmu, bmu, Wvar, bvar,
           Wu, bu, Ww, bw, Wb, bb, Wd1, bd1, Wd2, bd2, Wd3, bd3):
    src = edge_index[0]
    dst = edge_index[1]

    p, q = _pq(x, We1)

    srcr = src.reshape(NW, NCH, CH)
    dstr = dst.reshape(NW, NCH, CH)

    h1r, cnt = _sc_gather(p, q, srcr, dstr)
    msg = _mlp(h1r.reshape(E, BIG), be1.reshape(1, BIG),
               We2, be2.reshape(1, BIG), We3, be3.reshape(1, BIG))
    sums = _sc_scatter(msg.reshape(NW, EW, BIG), dstr)

    eps = jax.random.normal(jax.random.key(42), (N, HID), dtype=jnp.float32)
    mu, lv, z0, zk, ldj = _flow(
        sums, cnt, eps,
        Wmu, bmu.reshape(1, HID), Wvar, bvar.reshape(1, HID),
        Wu, bu.reshape(1, NF * HID), Ww, bw.reshape(1, NF * HID),
        Wb, bb.reshape(1, NF))

    return (zk, mu, lv, ldj.reshape(N), z0, zk)


# speedup vs baseline: 1.7040x; 1.0040x over previous
import functools

import jax
import jax.numpy as jnp
from jax import lax
from jax.experimental import pallas as pl
from jax.experimental.pallas import tpu as pltpu
from jax.experimental.pallas import tpu_sc as plsc

N = 10000
E = 320000
DIN = 128
BIG = 32
HID = 32
NF = 6

NC = 2
NS = 16
NW = NC * NS
EW = E // NW
CH = 125
NCH = EW // CH
CW = 16
SB = 1000
NSG = EW // SB
CPS = SB // CH
RB = 1250
RPT = 624
ZR = N // NS

_SC_MESH = dict(core_axis_name="c", subcore_axis_name="s",
                num_cores=NC, num_subcores=NS)
_SC_PARAMS = pltpu.CompilerParams(use_tc_tiling_on_sc=False)


def _drain(acc, out_hbm, cid, sid):
    last = N - (NS - 1) * RPT

    @pl.when(sid < NS - 1)
    def _():
        pltpu.sync_copy(acc.at[pl.ds(sid * RPT, RPT)],
                        out_hbm.at[cid, pl.ds(sid * RPT, RPT)])

    @pl.when(sid == NS - 1)
    def _():
        pltpu.sync_copy(acc.at[pl.ds((NS - 1) * RPT, last)],
                        out_hbm.at[cid, pl.ds((NS - 1) * RPT, last)])


def _zero_acc(zrow, acc, sid, width):
    def z(i, c):
        zrow[i, :] = jnp.zeros((width,), jnp.float32)
        return c
    lax.fori_loop(0, ZR, z, 0)
    pltpu.sync_copy(zrow, acc.at[pl.ds(sid * ZR, ZR)])


def _pq_body(x_ref, w1_ref, p_ref, q_ref):
    x = x_ref[...]
    wa = w1_ref[0:DIN, :] - w1_ref[DIN:2 * DIN, :]
    wb = w1_ref[DIN:2 * DIN, :]
    p_ref[...] = jnp.dot(x, wa, preferred_element_type=jnp.float32)
    q_ref[...] = jnp.dot(x, wb, preferred_element_type=jnp.float32)


def _pq(x, w1):
    nb = 10
    blk = N // nb
    return pl.pallas_call(
        _pq_body,
        grid=(nb,),
        in_specs=[
            pl.BlockSpec((blk, DIN), lambda i: (i, 0)),
            pl.BlockSpec((2 * DIN, BIG), lambda i: (0, 0)),
        ],
        out_specs=[
            pl.BlockSpec((blk, BIG), lambda i: (i, 0)),
            pl.BlockSpec((blk, BIG), lambda i: (i, 0)),
        ],
        out_shape=[
            jax.ShapeDtypeStruct((N, BIG), jnp.float32),
            jax.ShapeDtypeStruct((N, BIG), jnp.float32),
        ],
    )(x, w1)


def _sc_gather_body(p_hbm, q_hbm, srcg_hbm, dstg_hbm, dsts_hbm,
                    h1_hbm, cnt_hbm,
                    sidx, didx, didx_s, stage0, stage1, ones_v, zrow, acc_c,
                    sem_p0, sem_p1, sem_q, sem_c, sem_s0, sem_s1):
    cid = lax.axis_index("c")
    sid = lax.axis_index("s")
    wid = sid * NC + cid

    _zero_acc(zrow, acc_c, sid, CW)

    pltpu.sync_copy(srcg_hbm.at[wid], sidx)
    pltpu.sync_copy(dstg_hbm.at[wid], didx)
    pltpu.sync_copy(dsts_hbm.at[wid], didx_s)

    def fill_ones(i, c):
        ones_v[i, :] = jnp.full((CW,), 1.0, jnp.float32)
        return c
    lax.fori_loop(0, CH, fill_ones, 0)

    plsc.subcore_barrier()

    def fire_p(s, buf, psem):
        pltpu.async_copy(p_hbm.at[didx.at[s]], buf, psem)

    def wait_p(s, buf, psem):
        pltpu.make_async_copy(p_hbm.at[didx.at[s]], buf, psem).wait()

    def fire_q(s, buf):
        pltpu.async_copy(q_hbm.at[sidx.at[s]], buf, sem_q, add=True)

    def wait_q(s, buf):
        pltpu.make_async_copy(q_hbm.at[sidx.at[s]], buf, sem_q).wait()

    def fire_counts(s):
        for j in range(CPS):
            pltpu.async_copy(ones_v, acc_c.at[didx_s.at[s, j]],
                             sem_c, add=True)

    def wait_counts(s):
        for j in range(CPS):
            pltpu.make_async_copy(ones_v, acc_c.at[didx_s.at[s, j]],
                                  sem_c).wait()

    def store(s, buf, st_sem):
        pltpu.async_copy(buf, h1_hbm.at[wid, s], st_sem)

    def wait_store(buf, st_sem):
        pltpu.make_async_copy(buf, h1_hbm.at[wid, 0], st_sem).wait()

    fire_p(0, stage0, sem_p0)
    npair = NSG // 2

    def pair_loop(t, c):
        s0 = t * 2
        fire_counts(s0)

        @pl.when(t > 0)
        def _():
            wait_store(stage1, sem_s1)
        fire_p(s0 + 1, stage1, sem_p1)
        wait_p(s0, stage0, sem_p0)
        fire_q(s0, stage0)
        fire_counts(s0 + 1)
        wait_q(s0, stage0)
        store(s0, stage0, sem_s0)
        wait_p(s0 + 1, stage1, sem_p1)
        fire_q(s0 + 1, stage1)

        @pl.when(t + 1 < npair)
        def _():
            wait_store(stage0, sem_s0)
            fire_p(s0 + 2, stage0, sem_p0)
        wait_q(s0 + 1, stage1)
        store(s0 + 1, stage1, sem_s1)
        wait_counts(s0)
        wait_counts(s0 + 1)
        return c
    lax.fori_loop(0, npair, pair_loop, 0)
    wait_store(stage0, sem_s0)
    wait_store(stage1, sem_s1)

    plsc.subcore_barrier()
    _drain(acc_c, cnt_hbm, cid, sid)


def _sc_gather(p, q, srcr, dstr):
    mesh = plsc.VectorSubcoreMesh(**_SC_MESH)
    f = functools.partial(
        pl.kernel,
        mesh=mesh,
        compiler_params=_SC_PARAMS,
        out_type=[
            jax.ShapeDtypeStruct((NW, NSG, SB, BIG), jnp.float32),
            jax.ShapeDtypeStruct((NC, N, CW), jnp.float32),
        ],
        scratch_types=[
            pltpu.VMEM((NSG, SB), jnp.int32),
            pltpu.VMEM((NSG, SB), jnp.int32),
            pltpu.VMEM((NSG, CPS, CH), jnp.int32),
            pltpu.VMEM((SB, BIG), jnp.float32),
            pltpu.VMEM((SB, BIG), jnp.float32),
            pltpu.VMEM((CH, CW), jnp.float32),
            pltpu.VMEM((ZR, CW), jnp.float32),
            pltpu.VMEM_SHARED((N, CW), jnp.float32),
            pltpu.SemaphoreType.DMA,
            pltpu.SemaphoreType.DMA,
            pltpu.SemaphoreType.DMA,
            pltpu.SemaphoreType.DMA,
            pltpu.SemaphoreType.DMA,
            pltpu.SemaphoreType.DMA,
        ],
    )(_sc_gather_body)
    return f(p, q, srcr, dstr, dstr.reshape(NW, NSG, CPS, CH))


def _mlp_body(h_ref, b1_ref, w2_ref, b2_ref, w3_ref, b3_ref, o_ref):
    r = 128 // BIG
    eye = jnp.eye(r, dtype=jnp.float32)
    w2bd = jnp.kron(eye, w2_ref[...])
    w3bd = jnp.kron(eye, w3_ref[...])
    t = jnp.maximum(h_ref[...] + b1_ref[...], 0.0)
    t = jnp.maximum(
        jnp.dot(t, w2bd, preferred_element_type=jnp.float32) + b2_ref[...], 0.0)
    o_ref[...] = jnp.maximum(
        jnp.dot(t, w3bd, preferred_element_type=jnp.float32) + b3_ref[...], 0.0)


def _mlp(h1, b1, w2, b2, w3, b3):
    lw = 128
    e4 = E * BIG // lw
    be = 8000
    nb = e4 // be
    r = lw // BIG
    b1t = jnp.tile(b1, (1, r))
    b2t = jnp.tile(b2, (1, r))
    b3t = jnp.tile(b3, (1, r))
    out = pl.pallas_call(
        _mlp_body,
        grid=(nb,),
        in_specs=[
            pl.BlockSpec((be, lw), lambda i: (i, 0)),
            pl.BlockSpec((1, lw), lambda i: (0, 0)),
            pl.BlockSpec((BIG, BIG), lambda i: (0, 0)),
            pl.BlockSpec((1, lw), lambda i: (0, 0)),
            pl.BlockSpec((BIG, BIG), lambda i: (0, 0)),
            pl.BlockSpec((1, lw), lambda i: (0, 0)),
        ],
        out_specs=pl.BlockSpec((be, lw), lambda i: (i, 0)),
        out_shape=jax.ShapeDtypeStruct((e4, lw), jnp.float32),
    )(h1.reshape(e4, lw), b1t, w2, b2t, w3, b3t)
    return out.reshape(E, BIG)


def _sc_scatter_body(msg_hbm, dst_hbm, sum_hbm,
                     didx, mrows0, mrows1, zrow, acc_s,
                     sem_l0, sem_l1, sem_c0, sem_c1):
    cid = lax.axis_index("c")
    sid = lax.axis_index("s")
    wid = sid * NC + cid

    _zero_acc(zrow, acc_s, sid, BIG)

    pltpu.sync_copy(dst_hbm.at[wid], didx)
    plsc.subcore_barrier()

    def load(b, buf, lsem):
        pltpu.async_copy(msg_hbm.at[wid, b], buf, lsem)

    def wait_load(buf, lsem):
        pltpu.make_async_copy(msg_hbm.at[wid, 0], buf, lsem).wait()

    def fire_sc(b, buf, csem):
        for j in range(CPS):
            pltpu.async_copy(buf.at[pl.ds(j * CH, CH)],
                             acc_s.at[didx.at[b, j]], csem, add=True)

    def wait_sc(b, buf, csem):
        for j in range(CPS):
            pltpu.make_async_copy(buf.at[pl.ds(j * CH, CH)],
                                  acc_s.at[didx.at[b, j]], csem).wait()

    load(0, mrows0, sem_l0)

    def pair(t, c):
        b0 = t * 2

        @pl.when(t > 0)
        def _():
            wait_sc(b0 - 1, mrows1, sem_c1)

        @pl.when(b0 + 1 < NSG)
        def _():
            load(b0 + 1, mrows1, sem_l1)
        wait_load(mrows0, sem_l0)
        fire_sc(b0, mrows0, sem_c0)

        @pl.when(b0 + 2 < NSG)
        def _():
            wait_sc(b0, mrows0, sem_c0)
            load(b0 + 2, mrows0, sem_l0)

        @pl.when(b0 + 1 < NSG)
        def _():
            wait_load(mrows1, sem_l1)
            fire_sc(b0 + 1, mrows1, sem_c1)
        return c
    lax.fori_loop(0, (NSG + 1) // 2, pair, 0)
    wait_sc(NSG - 2, mrows0, sem_c0)
    wait_sc(NSG - 1, mrows1, sem_c1)

    plsc.subcore_barrier()
    _drain(acc_s, sum_hbm, cid, sid)


def _sc_scatter(msgr, dstr):
    mesh = plsc.VectorSubcoreMesh(**_SC_MESH)
    f = functools.partial(
        pl.kernel,
        mesh=mesh,
        compiler_params=_SC_PARAMS,
        out_type=jax.ShapeDtypeStruct((NC, N, BIG), jnp.float32),
        scratch_types=[
            pltpu.VMEM((NSG, CPS, CH), jnp.int32),
            pltpu.VMEM((SB, BIG), jnp.float32),
            pltpu.VMEM((SB, BIG), jnp.float32),
            pltpu.VMEM((ZR, BIG), jnp.float32),
            pltpu.VMEM_SHARED((N, BIG), jnp.float32),
            pltpu.SemaphoreType.DMA,
            pltpu.SemaphoreType.DMA,
            pltpu.SemaphoreType.DMA,
            pltpu.SemaphoreType.DMA,
        ],
    )(_sc_scatter_body)
    return f(msgr, dstr)


def _flow_body(s_ref, c_ref, eps_ref,
               wmu_ref, bmu_ref, wvar_ref, bvar_ref,
               wu_ref, bu_ref, ww_ref, bw_ref, wb_ref, bb_ref,
               mu_ref, lv_ref, z0_ref, zk_ref, ldj_ref):
    cnt = c_ref[0][:, 0:1] + c_ref[1][:, 0:1]
    h = (s_ref[0] + s_ref[1]) / jnp.maximum(cnt, 1.0)
    hT = jnp.transpose(h, (1, 0))

    def head(w_ref, b_ref):
        wT = jnp.transpose(w_ref[...], (1, 0))
        bT = jnp.transpose(b_ref[...], (1, 0))
        return jnp.dot(wT, hT, preferred_element_type=jnp.float32) + bT

    mu = head(wmu_ref, bmu_ref)
    lv = head(wvar_ref, bvar_ref)
    uu = head(wu_ref, bu_ref)
    ww = head(ww_ref, bw_ref)
    bf = head(wb_ref, bb_ref)

    epsT = jnp.transpose(eps_ref[...], (1, 0))
    z = mu + epsT * jnp.exp(0.5 * lv)
    mu_ref[...] = jnp.transpose(mu, (1, 0))
    lv_ref[...] = jnp.transpose(lv, (1, 0))
    z0_ref[...] = jnp.transpose(z, (1, 0))

    ldj = jnp.zeros_like(bf[0:1])
    for k in range(NF):
        uk = uu[k * HID:(k + 1) * HID]
        wk = ww[k * HID:(k + 1) * HID]
        bk = bf[k:k + 1]
        uw = jnp.sum(wk * uk, axis=0, keepdims=True)
        m_uw = -1.0 + jnp.logaddexp(uw, 0.0)
        wns = jnp.sum(wk * wk, axis=0, keepdims=True)
        u_hat = uk + ((m_uw - uw) / wns) * wk
        wzb = jnp.sum(wk * z, axis=0, keepdims=True) + bk
        t = jnp.tanh(wzb)
        z = z + u_hat * t
        wu_dot = jnp.sum(wk * u_hat, axis=0, keepdims=True)
        ldj = ldj + jnp.log(jnp.abs(1.0 + (1.0 - t * t) * wu_dot))

    zk_ref[...] = jnp.transpose(z, (1, 0))
    ldj_ref[...] = jnp.transpose(ldj, (1, 0))


def _flow(sums, cnts, eps, wmu, bmu, wvar, bvar, wu, bu, ww, bw, wb, bb):
    nb = 10
    blk = N // nb
    full = lambda r, c: pl.BlockSpec((r, c), lambda i: (0, 0))
    return pl.pallas_call(
        _flow_body,
        grid=(nb,),
        in_specs=[
            pl.BlockSpec((NC, blk, BIG), lambda i: (0, i, 0)),
            pl.BlockSpec((NC, blk, CW), lambda i: (0, i, 0)),
            pl.BlockSpec((blk, HID), lambda i: (i, 0)),
            full(BIG, HID), full(1, HID),
            full(BIG, HID), full(1, HID),
            full(BIG, NF * HID), full(1, NF * HID),
            full(BIG, NF * HID), full(1, NF * HID),
            full(BIG, NF), full(1, NF),
        ],
        out_specs=[
            pl.BlockSpec((blk, HID), lambda i: (i, 0)),
            pl.BlockSpec((blk, HID), lambda i: (i, 0)),
            pl.BlockSpec((blk, HID), lambda i: (i, 0)),
            pl.BlockSpec((blk, HID), lambda i: (i, 0)),
            pl.BlockSpec((blk, 1), lambda i: (i, 0)),
        ],
        out_shape=[
            jax.ShapeDtypeStruct((N, HID), jnp.float32),
            jax.ShapeDtypeStruct((N, HID), jnp.float32),
            jax.ShapeDtypeStruct((N, HID), jnp.float32),
            jax.ShapeDtypeStruct((N, HID), jnp.float32),
            jax.ShapeDtypeStruct((N, 1), jnp.float32),
        ],
    )(sums, cnts, eps, wmu, bmu, wvar, bvar, wu, bu, ww, bw, wb, bb)


def kernel(x, edge_index, We1, be1, We2, be2, We3, be3, Wmu, bmu, Wvar, bvar,
           Wu, bu, Ww, bw, Wb, bb, Wd1, bd1, Wd2, bd2, Wd3, bd3):
    src = edge_index[0]
    dst = edge_index[1]

    p, q = _pq(x, We1)

    srcr = src.reshape(NW, NSG, SB)
    dstr = dst.reshape(NW, NSG, SB)

    h1r, cnt = _sc_gather(p, q, srcr, dstr)
    msg = _mlp(h1r.reshape(E, BIG), be1.reshape(1, BIG),
               We2, be2.reshape(1, BIG), We3, be3.reshape(1, BIG))
    sums = _sc_scatter(msg.reshape(NW, NSG, SB, BIG),
                       dstr.reshape(NW, NSG, CPS, CH))

    eps = jax.random.normal(jax.random.key(42), (N, HID), dtype=jnp.float32)
    mu, lv, z0, zk, ldj = _flow(
        sums, cnt, eps,
        Wmu, bmu.reshape(1, HID), Wvar, bvar.reshape(1, HID),
        Wu, bu.reshape(1, NF * HID), Ww, bw.reshape(1, NF * HID),
        Wb, bb.reshape(1, NF))

    return (zk, mu, lv, ldj.reshape(N), z0, zk)
